# baseline (speedup 1.0000x reference)
import jax
import jax.numpy as jnp
from jax import lax
from jax.experimental import pallas as pl
from jax.experimental.pallas import tpu as pltpu
from jax.experimental.pallas import tpu_sc as plsc

NUM_MODELS = 1000000
D = 64
BATCH = 16384

NUM_CORES = 2
NUM_SUBCORES = 16
NUM_WORKERS = NUM_CORES * NUM_SUBCORES
B_PER_W = BATCH // NUM_WORKERS
IDX_CHUNK = 128
N_CHUNKS = B_PER_W // IDX_CHUNK


def _btd_kernel(i_hbm, j_hbm, k_hbm, u_hbm, v_hbm, ll_hbm, out_hbm,
                i_v, j_v, k_v, lli_v, u_rows, vj_rows, vk_rows, ll_rows,
                out_v, idx_sem, gat_sem):
    wid = lax.axis_index("s") * NUM_CORES + lax.axis_index("c")
    base = wid * B_PER_W

    idx_copies = [
        pltpu.async_copy(i_hbm.at[pl.ds(base, B_PER_W)], i_v, idx_sem),
        pltpu.async_copy(j_hbm.at[pl.ds(base, B_PER_W)], j_v, idx_sem),
        pltpu.async_copy(k_hbm.at[pl.ds(base, B_PER_W)], k_v, idx_sem),
    ]
    for cp in idx_copies:
        cp.wait()

    def shift_body(it, carry):
        sl = pl.ds(it * 16, 16)
        lli_v[sl] = lax.shift_right_logical(i_v[sl], 4)
        return carry

    lax.fori_loop(0, B_PER_W // 16, shift_body, 0)

    copies = []
    for c in range(N_CHUNKS):
        sl = pl.ds(c * IDX_CHUNK, IDX_CHUNK)
        copies.append(pltpu.async_copy(
            u_hbm.at[i_v.at[sl]], u_rows.at[sl], gat_sem))
        copies.append(pltpu.async_copy(
            v_hbm.at[j_v.at[sl]], vj_rows.at[sl], gat_sem))
        copies.append(pltpu.async_copy(
            v_hbm.at[k_v.at[sl]], vk_rows.at[sl], gat_sem))
        copies.append(pltpu.async_copy(
            ll_hbm.at[lli_v.at[sl]], ll_rows.at[sl], gat_sem))
    for cp in copies:
        cp.wait()

    lane = lax.iota(jnp.int32, 16)
    zeros_i = jnp.zeros((16,), jnp.int32)
    ones_i = jnp.full((16,), 1, jnp.int32)
    twos_i = jnp.full((16,), 2, jnp.int32)

    def body(it, carry):
        b0 = it * 16
        rows = b0 + lane
        sjv = jnp.zeros((16,), jnp.float32)
        skv = jnp.zeros((16,), jnp.float32)
        for d in range(D):
            col = jnp.full((16,), d, jnp.int32)
            u_d = plsc.load_gather(u_rows, [rows, col])
            sjv = sjv + u_d * plsc.load_gather(vj_rows, [rows, col])
            skv = skv + u_d * plsc.load_gather(vk_rows, [rows, col])
        ll16 = plsc.load_gather(ll_rows, [rows, i_v[pl.ds(b0, 16)] & 15])
        tie = ll16 + 0.5 * (sjv + skv)
        plsc.store_scatter(out_v, [rows, zeros_i], tie)
        plsc.store_scatter(out_v, [rows, ones_i], sjv)
        plsc.store_scatter(out_v, [rows, twos_i], skv)
        return carry

    lax.fori_loop(0, B_PER_W // 16, body, 0)

    pltpu.sync_copy(out_v, out_hbm.at[pl.ds(base, B_PER_W)])


@jax.jit
def kernel(i, j, k, u_weight, v_weight, log_lambda_weight):
    mesh = plsc.VectorSubcoreMesh(
        core_axis_name="c", subcore_axis_name="s",
        num_cores=NUM_CORES, num_subcores=NUM_SUBCORES)
    run = pl.kernel(
        _btd_kernel,
        out_type=jax.ShapeDtypeStruct((BATCH, 3), jnp.float32),
        mesh=mesh,
        compiler_params=pltpu.CompilerParams(
            needs_layout_passes=False, use_tc_tiling_on_sc=False),
        scratch_types=[
            pltpu.VMEM((B_PER_W,), jnp.int32),
            pltpu.VMEM((B_PER_W,), jnp.int32),
            pltpu.VMEM((B_PER_W,), jnp.int32),
            pltpu.VMEM((B_PER_W,), jnp.int32),
            pltpu.VMEM((B_PER_W, D), jnp.float32),
            pltpu.VMEM((B_PER_W, D), jnp.float32),
            pltpu.VMEM((B_PER_W, D), jnp.float32),
            pltpu.VMEM((B_PER_W, 16), jnp.float32),
            pltpu.VMEM((B_PER_W, 3), jnp.float32),
            pltpu.SemaphoreType.DMA,
            pltpu.SemaphoreType.DMA,
        ],
    )
    ll2d = log_lambda_weight.reshape(NUM_MODELS // 16, 16)
    return run(i, j, k, u_weight, v_weight, ll2d)


# speedup vs baseline: 1.0019x; 1.0019x over previous
import jax
import jax.numpy as jnp
from jax import lax
from jax.experimental import pallas as pl
from jax.experimental.pallas import tpu as pltpu
from jax.experimental.pallas import tpu_sc as plsc

NUM_MODELS = 1000000
D = 64
BATCH = 16384

NUM_CORES = 2
NUM_SUBCORES = 16
NUM_WORKERS = NUM_CORES * NUM_SUBCORES
B_PER_W = BATCH // NUM_WORKERS
HALF = B_PER_W // 2
IDX_CHUNK = 128
LL_ROWS = (NUM_MODELS + 127) // 128


def _btd_kernel(i_hbm, j_hbm, k_hbm, up_hbm, vp_hbm, llp_hbm,
                o0_hbm, o1_hbm, o2_hbm,
                i_v, j_v, k_v, pi_v, pj_v, pk_v, li_v,
                u_rows, vj_rows, vk_rows,
                t0_v, t1_v, t2_v, idx_sem, gat_sem):
    wid = lax.axis_index("s") * NUM_CORES + lax.axis_index("c")
    base = wid * B_PER_W

    idx_copies = [
        pltpu.async_copy(i_hbm.at[pl.ds(base, B_PER_W)], i_v, idx_sem),
        pltpu.async_copy(j_hbm.at[pl.ds(base, B_PER_W)], j_v, idx_sem),
        pltpu.async_copy(k_hbm.at[pl.ds(base, B_PER_W)], k_v, idx_sem),
    ]
    for cp in idx_copies:
        cp.wait()

    def shift_body(it, carry):
        sl = pl.ds(it * 16, 16)
        pi_v[sl] = lax.shift_right_logical(i_v[sl], 1)
        pj_v[sl] = lax.shift_right_logical(j_v[sl], 1)
        pk_v[sl] = lax.shift_right_logical(k_v[sl], 1)
        li_v[sl] = lax.shift_right_logical(i_v[sl], 7)
        return carry

    lax.fori_loop(0, B_PER_W // 16, shift_body, 0)

    lane = lax.iota(jnp.int32, 16)

    for h in range(2):
        hb = h * HALF
        copies = []
        for c in range(HALF // IDX_CHUNK):
            src = pl.ds(hb + c * IDX_CHUNK, IDX_CHUNK)
            dst = pl.ds(c * IDX_CHUNK, IDX_CHUNK)
            copies.append(pltpu.async_copy(
                up_hbm.at[pi_v.at[src]], u_rows.at[dst], gat_sem))
            copies.append(pltpu.async_copy(
                vp_hbm.at[pj_v.at[src]], vj_rows.at[dst], gat_sem))
            copies.append(pltpu.async_copy(
                vp_hbm.at[pk_v.at[src]], vk_rows.at[dst], gat_sem))
        for cp in copies:
            cp.wait()

        def dot_body(g, carry):
            rows = g * 16 + lane
            mu = i_v[pl.ds(hb + g * 16, 16)]
            mj = j_v[pl.ds(hb + g * 16, 16)]
            mk = k_v[pl.ds(hb + g * 16, 16)]
            cbu = (mu & 1) * D
            cbj = (mj & 1) * D
            cbk = (mk & 1) * D
            sjv = jnp.zeros((16,), jnp.float32)
            skv = jnp.zeros((16,), jnp.float32)
            for d in range(D):
                u_d = plsc.load_gather(u_rows, [rows, cbu + d])
                sjv = sjv + u_d * plsc.load_gather(vj_rows, [rows, cbj + d])
                skv = skv + u_d * plsc.load_gather(vk_rows, [rows, cbk + d])
            t1_v[pl.ds(hb + g * 16, 16)] = sjv
            t2_v[pl.ds(hb + g * 16, 16)] = skv
            return carry

        lax.fori_loop(0, HALF // 16, dot_body, 0)

        ll_copies = []
        for c in range(HALF // IDX_CHUNK):
            src = pl.ds(hb + c * IDX_CHUNK, IDX_CHUNK)
            dst = pl.ds(c * IDX_CHUNK, IDX_CHUNK)
            ll_copies.append(pltpu.async_copy(
                llp_hbm.at[li_v.at[src]], u_rows.at[dst], gat_sem))
        for cp in ll_copies:
            cp.wait()

        def tie_body(g, carry):
            rows = g * 16 + lane
            sl = pl.ds(hb + g * 16, 16)
            mu = i_v[sl]
            llv = plsc.load_gather(u_rows, [rows, mu & 127])
            t0_v[sl] = llv + 0.5 * (t1_v[sl] + t2_v[sl])
            return carry

        lax.fori_loop(0, HALF // 16, tie_body, 0)

    pltpu.sync_copy(t0_v, o0_hbm.at[pl.ds(base, B_PER_W)])
    pltpu.sync_copy(t1_v, o1_hbm.at[pl.ds(base, B_PER_W)])
    pltpu.sync_copy(t2_v, o2_hbm.at[pl.ds(base, B_PER_W)])


@jax.jit
def kernel(i, j, k, u_weight, v_weight, log_lambda_weight):
    mesh = plsc.VectorSubcoreMesh(
        core_axis_name="c", subcore_axis_name="s",
        num_cores=NUM_CORES, num_subcores=NUM_SUBCORES)
    out1d = jax.ShapeDtypeStruct((BATCH,), jnp.float32)
    run = pl.kernel(
        _btd_kernel,
        out_type=(out1d, out1d, out1d),
        mesh=mesh,
        compiler_params=pltpu.CompilerParams(
            needs_layout_passes=False, use_tc_tiling_on_sc=True),
        scratch_types=[
            pltpu.VMEM((B_PER_W,), jnp.int32),
            pltpu.VMEM((B_PER_W,), jnp.int32),
            pltpu.VMEM((B_PER_W,), jnp.int32),
            pltpu.VMEM((B_PER_W,), jnp.int32),
            pltpu.VMEM((B_PER_W,), jnp.int32),
            pltpu.VMEM((B_PER_W,), jnp.int32),
            pltpu.VMEM((B_PER_W,), jnp.int32),
            pltpu.VMEM((HALF, 2 * D), jnp.float32),
            pltpu.VMEM((HALF, 2 * D), jnp.float32),
            pltpu.VMEM((HALF, 2 * D), jnp.float32),
            pltpu.VMEM((B_PER_W,), jnp.float32),
            pltpu.VMEM((B_PER_W,), jnp.float32),
            pltpu.VMEM((B_PER_W,), jnp.float32),
            pltpu.SemaphoreType.DMA,
            pltpu.SemaphoreType.DMA,
        ],
    )
    up = u_weight.reshape(NUM_MODELS // 2, 2 * D)
    vp = v_weight.reshape(NUM_MODELS // 2, 2 * D)
    llp = jnp.pad(log_lambda_weight.reshape(-1),
                  (0, LL_ROWS * 128 - NUM_MODELS)).reshape(LL_ROWS, 128)
    t0, t1, t2 = run(i, j, k, up, vp, llp)
    return jnp.stack([t0, t1, t2], axis=1)
